# baseline (speedup 1.0000x reference)
import functools

import jax
import jax.numpy as jnp
from jax import lax
from jax.experimental import pallas as pl
from jax.experimental.pallas import tpu as pltpu
from jax.experimental.pallas import tpu_sc as plsc

_ROWS, _COLS = 32768, 2048
_TC_ROWS = 24576
_CH = 256
_NBUF = 8
_NC, _NS, _L = 2, 16, 16
_NW = _NC * _NS
_SC_CH = 16
_UNROLL = 8

_I32MAX = jnp.iinfo(jnp.int32).max


def _tc_body(x_ref, val_ref, idx_ref, *, blk):
    i = pl.program_id(0)
    xb = x_ref[...]
    sq = jnp.sum(xb * xb, axis=1, keepdims=True)
    bv = jnp.max(sq)

    @pl.when(i == 0)
    def _():
        val_ref[0, 0] = -jnp.inf
        idx_ref[0, 0] = 0

    @pl.when(bv > val_ref[0, 0])
    def _():
        val_ref[0, 0] = bv
        row_ids = jax.lax.broadcasted_iota(jnp.int32, (blk, 1), 0)
        idx_ref[0, 0] = i * blk + jnp.min(jnp.where(sq == bv, row_ids, blk))


def _tc_scan(x):
    blk = 1024
    nsteps = _TC_ROWS // blk
    return pl.pallas_call(
        functools.partial(_tc_body, blk=blk),
        grid=(nsteps,),
        in_specs=[pl.BlockSpec((blk, _COLS), lambda i: (i, 0))],
        out_specs=[
            pl.BlockSpec(memory_space=pltpu.SMEM),
            pl.BlockSpec(memory_space=pltpu.SMEM),
        ],
        out_shape=[
            jax.ShapeDtypeStruct((1, 1), jnp.float32),
            jax.ShapeDtypeStruct((1, 1), jnp.int32),
        ],
    )(x)


def _sc_scan(x):
    sc_rows = _ROWS - _TC_ROWS
    rows_w = sc_rows // _NW
    nch = rows_w // _SC_CH
    mesh = plsc.VectorSubcoreMesh(core_axis_name="c", subcore_axis_name="s")

    @functools.partial(
        pl.kernel,
        mesh=mesh,
        out_type=[
            jax.ShapeDtypeStruct((_NW, _L), jnp.float32),
            jax.ShapeDtypeStruct((_NW, _L), jnp.int32),
        ],
        scratch_types=[
            pltpu.VMEM((2, _SC_CH, _COLS), jnp.float32),
            pltpu.VMEM((_L,), jnp.float32),
            pltpu.VMEM((_L,), jnp.int32),
            pltpu.SemaphoreType.DMA((2,)),
        ],
        compiler_params=pltpu.CompilerParams(needs_layout_passes=False),
    )
    def k(x_hbm, val_out, idx_out, buf, stage_v, stage_i, sems):
        wid = lax.axis_index("s") * _NC + lax.axis_index("c")
        base = _TC_ROWS + wid * rows_w

        def copy(slot, ch):
            return pltpu.make_async_copy(
                x_hbm.at[pl.ds(base + ch * _SC_CH, _SC_CH), :],
                buf.at[slot],
                sems.at[slot],
            )

        copy(0, 0).start()
        copy(1, 1).start()

        def chunk_pair(p, carry):
            for b in range(2):
                ch = 2 * p + b
                copy(b, ch).wait()

                def row_body(r, c2):
                    def col_body(j0, acc):
                        for u in range(_UNROLL):
                            v = buf[b, r, pl.ds((j0 * _UNROLL + u) * _L, _L)]
                            acc = acc + v * v
                        return acc

                    acc = lax.fori_loop(
                        0, _COLS // (_L * _UNROLL), col_body,
                        jnp.zeros((_L,), jnp.float32))
                    ssum = jnp.sum(acc)
                    bv, bi = c2
                    better = ssum > bv
                    nbv = jnp.where(better, ssum, bv)
                    nbi = jnp.where(better, base + ch * _SC_CH + r, bi)
                    return (nbv, nbi)

                carry = lax.fori_loop(0, _SC_CH, row_body, carry)

                @pl.when(ch + 2 < nch)
                def _():
                    copy(b, ch + 2).start()

            return carry

        bv, bi = lax.fori_loop(0, nch // 2, chunk_pair,
                               (-jnp.inf, jnp.int32(0)))
        stage_v[...] = jnp.full((_L,), bv, jnp.float32)
        stage_i[...] = jnp.full((_L,), bi, jnp.int32)
        pltpu.sync_copy(stage_v, val_out.at[wid])
        pltpu.sync_copy(stage_i, idx_out.at[wid])

    return k(x)


def _combine_body(x_hbm, tcv_ref, tci_ref, scv_ref, sci_ref, o_ref, sem,
                  bi_ref):
    scv = scv_ref[...]
    sci = sci_ref[...]
    tcv = tcv_ref[0, 0]
    m_sc = jnp.max(scv)
    bi_sc = jnp.min(jnp.where(scv == m_sc, sci, _I32MAX))
    bi_ref[0] = jnp.where(tcv >= m_sc, tci_ref[0, 0], bi_sc)
    pltpu.make_async_copy(
        x_hbm.at[pl.ds(bi_ref[0], 1), :], o_ref, sem).start()
    pltpu.make_async_copy(
        x_hbm.at[pl.ds(bi_ref[0], 1), :], o_ref, sem).wait()


def _combine(x, tcv, tci, scv, sci):
    return pl.pallas_call(
        _combine_body,
        in_specs=[
            pl.BlockSpec(memory_space=pl.ANY),
            pl.BlockSpec(memory_space=pltpu.SMEM),
            pl.BlockSpec(memory_space=pltpu.SMEM),
            pl.BlockSpec((_NW, _L), lambda: (0, 0)),
            pl.BlockSpec((_NW, _L), lambda: (0, 0)),
        ],
        out_specs=pl.BlockSpec((1, _COLS), lambda: (0, 0)),
        out_shape=jax.ShapeDtypeStruct((1, _COLS), jnp.float32),
        scratch_shapes=[
            pltpu.SemaphoreType.DMA,
            pltpu.SMEM((1,), jnp.int32),
        ],
    )(x, tcv, tci, scv, sci)


def kernel(x):
    scv, sci = _sc_scan(x)
    tcv, tci = _tc_scan(x)
    return _combine(x, tcv, tci, scv, sci)


# speedup vs baseline: 1.0144x; 1.0144x over previous
import functools

import jax
import jax.numpy as jnp
from jax import lax
from jax.experimental import pallas as pl
from jax.experimental.pallas import tpu as pltpu
from jax.experimental.pallas import tpu_sc as plsc

_ROWS, _COLS = 32768, 2048
_TC_ROWS = 30720
_CH = 256
_NBUF = 8
_NC, _NS, _L = 2, 16, 16
_NW = _NC * _NS
_SC_CH = 16
_UNROLL = 8

_I32MAX = jnp.iinfo(jnp.int32).max


def _tc_body(x_hbm, val_ref, idx_ref, buf, sems, *, nsteps):
    i = pl.program_id(0)

    def copy(slot, step):
        return pltpu.make_async_copy(
            x_hbm.at[pl.ds(step * _CH, _CH), :],
            buf.at[slot],
            sems.at[slot],
        )

    @pl.when(i == 0)
    def _():
        val_ref[0, 0] = -jnp.inf
        idx_ref[0, 0] = 0
        for k in range(_NBUF):
            copy(k, k).start()

    slot = lax.rem(i, _NBUF)
    copy(slot, i).wait()
    xb = buf[slot]
    sq = jnp.sum(xb * xb, axis=1, keepdims=True)
    bv = jnp.max(sq)

    @pl.when(i + _NBUF < nsteps)
    def _():
        copy(slot, i + _NBUF).start()

    @pl.when(bv > val_ref[0, 0])
    def _():
        val_ref[0, 0] = bv
        row_ids = jax.lax.broadcasted_iota(jnp.int32, (_CH, 1), 0)
        idx_ref[0, 0] = i * _CH + jnp.min(jnp.where(sq == bv, row_ids, _CH))


def _tc_scan(x):
    nsteps = _TC_ROWS // _CH
    return pl.pallas_call(
        functools.partial(_tc_body, nsteps=nsteps),
        grid=(nsteps,),
        in_specs=[pl.BlockSpec(memory_space=pl.ANY)],
        out_specs=[
            pl.BlockSpec(memory_space=pltpu.SMEM),
            pl.BlockSpec(memory_space=pltpu.SMEM),
        ],
        out_shape=[
            jax.ShapeDtypeStruct((1, 1), jnp.float32),
            jax.ShapeDtypeStruct((1, 1), jnp.int32),
        ],
        scratch_shapes=[
            pltpu.VMEM((_NBUF, _CH, _COLS), jnp.float32),
            pltpu.SemaphoreType.DMA((_NBUF,)),
        ],
        compiler_params=pltpu.CompilerParams(
            dimension_semantics=("arbitrary",),
        ),
    )(x)


def _sc_scan(x):
    sc_rows = _ROWS - _TC_ROWS
    rows_w = sc_rows // _NW
    nch = rows_w // _SC_CH
    mesh = plsc.VectorSubcoreMesh(core_axis_name="c", subcore_axis_name="s")

    @functools.partial(
        pl.kernel,
        mesh=mesh,
        out_type=[
            jax.ShapeDtypeStruct((_NW, _L), jnp.float32),
            jax.ShapeDtypeStruct((_NW, _L), jnp.int32),
        ],
        scratch_types=[
            pltpu.VMEM((2, _SC_CH, _COLS), jnp.float32),
            pltpu.VMEM((_L,), jnp.float32),
            pltpu.VMEM((_L,), jnp.int32),
            pltpu.SemaphoreType.DMA((2,)),
        ],
        compiler_params=pltpu.CompilerParams(needs_layout_passes=False),
    )
    def k(x_hbm, val_out, idx_out, buf, stage_v, stage_i, sems):
        wid = lax.axis_index("s") * _NC + lax.axis_index("c")
        base = _TC_ROWS + wid * rows_w

        def copy(slot, ch):
            return pltpu.make_async_copy(
                x_hbm.at[pl.ds(base + ch * _SC_CH, _SC_CH), :],
                buf.at[slot],
                sems.at[slot],
            )

        copy(0, 0).start()
        copy(1, 1).start()

        def chunk_pair(p, carry):
            for b in range(2):
                ch = 2 * p + b
                copy(b, ch).wait()

                def row_body(r, c2):
                    def col_body(j0, acc):
                        for u in range(_UNROLL):
                            v = buf[b, r, pl.ds((j0 * _UNROLL + u) * _L, _L)]
                            acc = acc + v * v
                        return acc

                    acc = lax.fori_loop(
                        0, _COLS // (_L * _UNROLL), col_body,
                        jnp.zeros((_L,), jnp.float32))
                    ssum = jnp.sum(acc)
                    bv, bi = c2
                    better = ssum > bv
                    nbv = jnp.where(better, ssum, bv)
                    nbi = jnp.where(better, base + ch * _SC_CH + r, bi)
                    return (nbv, nbi)

                carry = lax.fori_loop(0, _SC_CH, row_body, carry)

                @pl.when(ch + 2 < nch)
                def _():
                    copy(b, ch + 2).start()

            return carry

        bv, bi = lax.fori_loop(0, nch // 2, chunk_pair,
                               (-jnp.inf, jnp.int32(0)))
        stage_v[...] = jnp.full((_L,), bv, jnp.float32)
        stage_i[...] = jnp.full((_L,), bi, jnp.int32)
        pltpu.sync_copy(stage_v, val_out.at[wid])
        pltpu.sync_copy(stage_i, idx_out.at[wid])

    return k(x)


def _combine_body(x_hbm, tcv_ref, tci_ref, scv_ref, sci_ref, o_ref, sem,
                  bi_ref):
    scv = scv_ref[...]
    sci = sci_ref[...]
    tcv = tcv_ref[0, 0]
    m_sc = jnp.max(scv)
    bi_sc = jnp.min(jnp.where(scv == m_sc, sci, _I32MAX))
    bi_ref[0] = jnp.where(tcv >= m_sc, tci_ref[0, 0], bi_sc)
    pltpu.make_async_copy(
        x_hbm.at[pl.ds(bi_ref[0], 1), :], o_ref, sem).start()
    pltpu.make_async_copy(
        x_hbm.at[pl.ds(bi_ref[0], 1), :], o_ref, sem).wait()


def _combine(x, tcv, tci, scv, sci):
    return pl.pallas_call(
        _combine_body,
        in_specs=[
            pl.BlockSpec(memory_space=pl.ANY),
            pl.BlockSpec(memory_space=pltpu.SMEM),
            pl.BlockSpec(memory_space=pltpu.SMEM),
            pl.BlockSpec((_NW, _L), lambda: (0, 0)),
            pl.BlockSpec((_NW, _L), lambda: (0, 0)),
        ],
        out_specs=pl.BlockSpec((1, _COLS), lambda: (0, 0)),
        out_shape=jax.ShapeDtypeStruct((1, _COLS), jnp.float32),
        scratch_shapes=[
            pltpu.SemaphoreType.DMA,
            pltpu.SMEM((1,), jnp.int32),
        ],
    )(x, tcv, tci, scv, sci)


def kernel(x):
    scv, sci = _sc_scan(x)
    tcv, tci = _tc_scan(x)
    return _combine(x, tcv, tci, scv, sci)


# speedup vs baseline: 1.2053x; 1.1882x over previous
import jax
import jax.numpy as jnp
from jax.experimental import pallas as pl
from jax.experimental.pallas import tpu as pltpu

_CH = 512
_NBUF = 8


def _body(x_hbm, o_ref, buf, best, sems, *, nsteps):
    i = pl.program_id(0)

    def copy(slot, step):
        return pltpu.make_async_copy(
            x_hbm.at[pl.ds(step * _CH, _CH), :],
            buf.at[slot],
            sems.at[slot],
        )

    @pl.when(i == 0)
    def _():
        best[0] = -jnp.inf
        for k in range(_NBUF):
            copy(k, k).start()

    slot = jax.lax.rem(i, _NBUF)
    copy(slot, i).wait()
    xb = buf[slot]
    sq = jnp.sum(xb * xb, axis=1, keepdims=True)
    bv = jnp.max(sq)

    @pl.when(i + _NBUF < nsteps)
    def _():
        copy(slot, i + _NBUF).start()

    @pl.when(bv > best[0])
    def _():
        best[0] = bv
        row_ids = jax.lax.broadcasted_iota(jnp.int32, (_CH, 1), 0)
        bi = jnp.min(jnp.where(sq == bv, row_ids, _CH))
        onehot = (row_ids == bi).astype(xb.dtype)
        o_ref[...] = jnp.sum(xb * onehot, axis=0, keepdims=True)


def kernel(x):
    rows, cols = x.shape
    nsteps = rows // _CH
    import functools
    return pl.pallas_call(
        functools.partial(_body, nsteps=nsteps),
        grid=(nsteps,),
        in_specs=[pl.BlockSpec(memory_space=pl.ANY)],
        out_specs=pl.BlockSpec((1, cols), lambda i: (0, 0)),
        out_shape=jax.ShapeDtypeStruct((1, cols), x.dtype),
        scratch_shapes=[
            pltpu.VMEM((_NBUF, _CH, cols), jnp.float32),
            pltpu.SMEM((1,), jnp.float32),
            pltpu.SemaphoreType.DMA((_NBUF,)),
        ],
        compiler_params=pltpu.CompilerParams(
            dimension_semantics=("arbitrary",),
        ),
    )(x)


# speedup vs baseline: 1.2198x; 1.0120x over previous
import jax
import jax.numpy as jnp
from jax.experimental import pallas as pl
from jax.experimental.pallas import tpu as pltpu

_CH = 256
_NBUF = 12


def _body(x_hbm, o_ref, buf, best, sems, *, nsteps):
    i = pl.program_id(0)

    def copy(slot, step):
        return pltpu.make_async_copy(
            x_hbm.at[pl.ds(step * _CH, _CH), :],
            buf.at[slot],
            sems.at[slot],
        )

    @pl.when(i == 0)
    def _():
        best[0] = -jnp.inf
        for k in range(_NBUF):
            copy(k, k).start()

    slot = jax.lax.rem(i, _NBUF)
    copy(slot, i).wait()
    xb = buf[slot]
    sq = jnp.sum(xb * xb, axis=1, keepdims=True)
    bv = jnp.max(sq)

    @pl.when(i + _NBUF < nsteps)
    def _():
        copy(slot, i + _NBUF).start()

    @pl.when(bv > best[0])
    def _():
        best[0] = bv
        row_ids = jax.lax.broadcasted_iota(jnp.int32, (_CH, 1), 0)
        bi = jnp.min(jnp.where(sq == bv, row_ids, _CH))
        onehot = (row_ids == bi).astype(xb.dtype)
        o_ref[...] = jnp.sum(xb * onehot, axis=0, keepdims=True)


def kernel(x):
    rows, cols = x.shape
    nsteps = rows // _CH
    import functools
    return pl.pallas_call(
        functools.partial(_body, nsteps=nsteps),
        grid=(nsteps,),
        in_specs=[pl.BlockSpec(memory_space=pl.ANY)],
        out_specs=pl.BlockSpec((1, cols), lambda i: (0, 0)),
        out_shape=jax.ShapeDtypeStruct((1, cols), x.dtype),
        scratch_shapes=[
            pltpu.VMEM((_NBUF, _CH, cols), jnp.float32),
            pltpu.SMEM((1,), jnp.float32),
            pltpu.SemaphoreType.DMA((_NBUF,)),
        ],
        compiler_params=pltpu.CompilerParams(
            dimension_semantics=("arbitrary",),
        ),
    )(x)


# speedup vs baseline: 1.2585x; 1.0317x over previous
import jax
import jax.numpy as jnp
from jax.experimental import pallas as pl
from jax.experimental.pallas import tpu as pltpu

_CH = 256
_NBUF = 8


def _body(x_hbm, o_ref, buf, best, sems, *, nsteps):
    i = pl.program_id(0)

    def copy(slot, step):
        return pltpu.make_async_copy(
            x_hbm.at[pl.ds(step * _CH, _CH), :],
            buf.at[slot],
            sems.at[slot],
        )

    @pl.when(i == 0)
    def _():
        best[0] = -jnp.inf
        for k in range(_NBUF):
            copy(k, k).start()

    slot = jax.lax.rem(i, _NBUF)
    copy(slot, i).wait()
    xb = buf[slot]
    sq = jnp.sum(xb * xb, axis=1, keepdims=True)
    bv = jnp.max(sq)

    @pl.when(i + _NBUF < nsteps)
    def _():
        copy(slot, i + _NBUF).start()

    @pl.when(bv > best[0])
    def _():
        best[0] = bv
        row_ids = jax.lax.broadcasted_iota(jnp.int32, (_CH, 1), 0)
        bi = jnp.min(jnp.where(sq == bv, row_ids, _CH))
        onehot = (row_ids == bi).astype(xb.dtype)
        o_ref[...] = jnp.sum(xb * onehot, axis=0, keepdims=True)


def kernel(x):
    rows, cols = x.shape
    nsteps = rows // _CH
    import functools
    return pl.pallas_call(
        functools.partial(_body, nsteps=nsteps),
        grid=(nsteps,),
        in_specs=[pl.BlockSpec(memory_space=pl.ANY)],
        out_specs=pl.BlockSpec((1, cols), lambda i: (0, 0)),
        out_shape=jax.ShapeDtypeStruct((1, cols), x.dtype),
        scratch_shapes=[
            pltpu.VMEM((_NBUF, _CH, cols), jnp.float32),
            pltpu.SMEM((1,), jnp.float32),
            pltpu.SemaphoreType.DMA((_NBUF,)),
        ],
        compiler_params=pltpu.CompilerParams(
            dimension_semantics=("arbitrary",),
        ),
    )(x)
